# baseline (speedup 1.0000x reference)
import jax
import jax.numpy as jnp
from jax import lax
from jax.experimental import pallas as pl
from jax.experimental.pallas import tpu as pltpu

_B, _N, _L = 16, 1024, 8
_CH = 1024


_RPS = 4


def _one_row(vals, labs):
    count = jnp.zeros((_N,), jnp.float32)
    for c in range(_N // _CH):
        vi = vals[c * _CH:(c + 1) * _CH][:, None]
        li = labs[c * _CH:(c + 1) * _CH][:, None]
        eq = (vi == vals[None, :]) & (li == labs[None, :])
        count = count + jnp.sum(eq.astype(jnp.float32), axis=0)

    gids = lax.broadcasted_iota(jnp.int32, (_L, 1), 0).astype(jnp.float32)
    m = labs[None, :] == gids
    gsize = jnp.sum(jnp.where(m, 1.0, 0.0), axis=1)
    gsum = jnp.sum(jnp.where(m, vals[None, :], 0.0), axis=1)
    gmax = jnp.max(jnp.where(m, count[None, :], -1.0), axis=1)
    cand = m & (count[None, :] == gmax[:, None])
    mode = jnp.min(jnp.where(cand, vals[None, :], jnp.inf), axis=1)
    contrib = jnp.where(gsize > 0, gsum - gsize * mode, 0.0)
    return jnp.sum(contrib)


def _row_body(vals_ref, labs_ref, out_ref):
    b = pl.program_id(0)
    total = jnp.float32(0.0)
    for r in range(_RPS):
        total = total + _one_row(vals_ref[r, 0, :], labs_ref[r, 0, :])

    @pl.when(b == 0)
    def _():
        out_ref[0, 0] = jnp.float32(0.0)

    out_ref[0, 0] += total / jnp.float32(_B * _N)


def kernel(inputs, targets):
    out = pl.pallas_call(
        _row_body,
        grid=(_B // _RPS,),
        in_specs=[
            pl.BlockSpec((_RPS, 1, _N), lambda b: (b, 0, 0)),
            pl.BlockSpec((_RPS, 1, _N), lambda b: (b, 0, 0)),
        ],
        out_specs=pl.BlockSpec((1, 1), lambda b: (0, 0), memory_space=pltpu.SMEM),
        out_shape=jax.ShapeDtypeStruct((1, 1), jnp.float32),
    )(inputs.reshape(_B, 1, _N), targets.reshape(_B, 1, _N))
    return out[0, 0]


# speedup vs baseline: 1.1465x; 1.1465x over previous
import jax
import jax.numpy as jnp
from jax import lax
from jax.experimental import pallas as pl
from jax.experimental.pallas import tpu as pltpu

_B, _N, _L = 16, 1024, 8
_CH = 1024


_RPS = 2


def _one_row(vals, labs):
    count = jnp.zeros((_N,), jnp.float32)
    for c in range(_N // _CH):
        vi = vals[c * _CH:(c + 1) * _CH][:, None]
        li = labs[c * _CH:(c + 1) * _CH][:, None]
        eq = (vi == vals[None, :]) & (li == labs[None, :])
        count = count + jnp.sum(eq.astype(jnp.float32), axis=0)

    gids = lax.broadcasted_iota(jnp.int32, (_L, 1), 0).astype(jnp.float32)
    m = labs[None, :] == gids
    gsize = jnp.sum(jnp.where(m, 1.0, 0.0), axis=1)
    gsum = jnp.sum(jnp.where(m, vals[None, :], 0.0), axis=1)
    gmax = jnp.max(jnp.where(m, count[None, :], -1.0), axis=1)
    cand = m & (count[None, :] == gmax[:, None])
    mode = jnp.min(jnp.where(cand, vals[None, :], jnp.inf), axis=1)
    contrib = jnp.where(gsize > 0, gsum - gsize * mode, 0.0)
    return jnp.sum(contrib)


def _row_body(vals_ref, labs_ref, out_ref):
    b = pl.program_id(0)
    total = jnp.float32(0.0)
    for r in range(_RPS):
        total = total + _one_row(vals_ref[r, 0, :], labs_ref[r, 0, :])

    @pl.when(b == 0)
    def _():
        out_ref[0, 0] = jnp.float32(0.0)

    out_ref[0, 0] += total / jnp.float32(_B * _N)


def kernel(inputs, targets):
    out = pl.pallas_call(
        _row_body,
        grid=(_B // _RPS,),
        in_specs=[
            pl.BlockSpec((_RPS, 1, _N), lambda b: (b, 0, 0)),
            pl.BlockSpec((_RPS, 1, _N), lambda b: (b, 0, 0)),
        ],
        out_specs=pl.BlockSpec((1, 1), lambda b: (0, 0), memory_space=pltpu.SMEM),
        out_shape=jax.ShapeDtypeStruct((1, 1), jnp.float32),
    )(inputs.reshape(_B, 1, _N), targets.reshape(_B, 1, _N))
    return out[0, 0]
